# baseline (speedup 1.0000x reference)
import functools

import jax
import jax.numpy as jnp
from jax import lax
from jax.experimental import pallas as pl
from jax.experimental.pallas import tpu as pltpu
from jax.experimental.pallas import tpu_sc as plsc

HIDDEN = 512
BATCH = 16384
_NC = 2
_NS = 16
_NW = _NC * _NS
_B_PER_W = BATCH // _NW
_CHUNK = 64
_NCHUNK = _B_PER_W // _CHUNK
_NBUF = 3


def _make_embed():
    mesh = plsc.VectorSubcoreMesh(core_axis_name="c", subcore_axis_name="s")

    @functools.partial(
        pl.kernel,
        mesh=mesh,
        out_type=jax.ShapeDtypeStruct((BATCH, HIDDEN), jnp.float32),
        scratch_types=[
            pltpu.VMEM((_B_PER_W,), jnp.int32),
            pltpu.VMEM((_NBUF, _CHUNK, HIDDEN), jnp.float32),
            pltpu.SemaphoreType.DMA((_NBUF,)),
            pltpu.SemaphoreType.DMA((_NBUF,)),
        ],
    )
    def embed(table_hbm, idx_hbm, out_hbm, idx_v, rows_v, sem_g, sem_s):
        wid = lax.axis_index("s") * _NC + lax.axis_index("c")
        base = wid * _B_PER_W
        pltpu.sync_copy(idx_hbm.at[pl.ds(base, _B_PER_W)], idx_v)

        gathers = [None] * _NCHUNK
        scatters = [None] * _NCHUNK

        def start_gather(c):
            b = c % _NBUF
            gathers[c] = pltpu.async_copy(
                table_hbm.at[idx_v.at[pl.ds(c * _CHUNK, _CHUNK)]],
                rows_v.at[b], sem_g.at[b])

        for c in range(min(_NBUF, _NCHUNK)):
            start_gather(c)
        for c in range(_NCHUNK):
            b = c % _NBUF
            gathers[c].wait()
            scatters[c] = pltpu.async_copy(
                rows_v.at[b], out_hbm.at[pl.ds(base + c * _CHUNK, _CHUNK)],
                sem_s.at[b])
            nxt = c + _NBUF
            if nxt < _NCHUNK:
                scatters[nxt - _NBUF].wait()
                start_gather(nxt)
        for c in range(max(0, _NCHUNK - _NBUF), _NCHUNK):
            scatters[c].wait()

    return embed


_embed = _make_embed()


def kernel(domain_ids, embed_weight):
    ids = domain_ids.astype(jnp.int32)
    return _embed(embed_weight, ids)


# speedup vs baseline: 7.9983x; 7.9983x over previous
import functools

import jax
import jax.numpy as jnp
from jax import lax
from jax.experimental import pallas as pl
from jax.experimental.pallas import tpu as pltpu
from jax.experimental.pallas import tpu_sc as plsc

HIDDEN = 512
BATCH = 16384
_NC = 2
_NS = 16
_NW = _NC * _NS
_B_PER_W = BATCH // _NW
_CHUNK = 64
_NCHUNK = _B_PER_W // _CHUNK
_NBUF = 3
_L = 16
_K = 16


def _make_embed():
    mesh = plsc.VectorSubcoreMesh(core_axis_name="c", subcore_axis_name="s")

    @functools.partial(
        pl.kernel,
        mesh=mesh,
        out_type=jax.ShapeDtypeStruct((BATCH, HIDDEN), jnp.float32),
        scratch_types=[
            pltpu.HBM((_NW * _K * 2, HIDDEN), jnp.float32),
            pltpu.VMEM((_B_PER_W,), jnp.int32),
            pltpu.VMEM((2, HIDDEN), jnp.float32),
            pltpu.VMEM((_NBUF, _CHUNK, HIDDEN), jnp.float32),
            pltpu.SemaphoreType.DMA((_NBUF,)),
            pltpu.SemaphoreType.DMA((_NBUF,)),
            pltpu.SemaphoreType.DMA,
        ],
    )
    def embed(table_hbm, idx_hbm, out_hbm, rep_hbm, idx_v, tab_v, rows_v,
              sem_g, sem_s, sem_r):
        wid = lax.axis_index("s") * _NC + lax.axis_index("c")
        base = wid * _B_PER_W

        pltpu.sync_copy(table_hbm, tab_v)
        rep_writes = [
            pltpu.async_copy(
                tab_v, rep_hbm.at[pl.ds((wid * _K + r) * 2, 2)], sem_r)
            for r in range(_K)]

        pltpu.sync_copy(idx_hbm.at[pl.ds(base, _B_PER_W)], idx_v)
        lane_off = lax.iota(jnp.int32, _L) * 2 + wid * (_K * 2)
        for g in range(_B_PER_W // _L):
            idx_v[pl.ds(g * _L, _L)] = idx_v[pl.ds(g * _L, _L)] + lane_off
        for w in rep_writes:
            w.wait()
        plsc.subcore_barrier()

        gathers = [None] * _NCHUNK
        scatters = [None] * _NCHUNK

        def start_gather(c):
            b = c % _NBUF
            gathers[c] = pltpu.async_copy(
                rep_hbm.at[idx_v.at[pl.ds(c * _CHUNK, _CHUNK)]],
                rows_v.at[b], sem_g.at[b])

        for c in range(min(_NBUF, _NCHUNK)):
            start_gather(c)
        for c in range(_NCHUNK):
            b = c % _NBUF
            gathers[c].wait()
            scatters[c] = pltpu.async_copy(
                rows_v.at[b], out_hbm.at[pl.ds(base + c * _CHUNK, _CHUNK)],
                sem_s.at[b])
            nxt = c + _NBUF
            if nxt < _NCHUNK:
                scatters[nxt - _NBUF].wait()
                start_gather(nxt)
        for c in range(max(0, _NCHUNK - _NBUF), _NCHUNK):
            scatters[c].wait()

    return embed


_embed = _make_embed()


def kernel(domain_ids, embed_weight):
    ids = domain_ids.astype(jnp.int32)
    return _embed(embed_weight, ids)


# speedup vs baseline: 9.4108x; 1.1766x over previous
import functools

import jax
import jax.numpy as jnp
from jax import lax
from jax.experimental import pallas as pl
from jax.experimental.pallas import tpu as pltpu
from jax.experimental.pallas import tpu_sc as plsc

HIDDEN = 512
BATCH = 16384
_NC = 2
_NS = 16
_NW = _NC * _NS
_B_PER_W = BATCH // _NW
_CHUNK = 64
_NCHUNK = _B_PER_W // _CHUNK
_NBUF = 3
_L = 16
_DC = HIDDEN // _L
_HALF = _DC // 2


def _make_embed():
    mesh = plsc.VectorSubcoreMesh(core_axis_name="c", subcore_axis_name="s")

    @functools.partial(
        pl.kernel,
        mesh=mesh,
        out_type=jax.ShapeDtypeStruct((BATCH, HIDDEN), jnp.float32),
        scratch_types=[
            pltpu.VMEM((_B_PER_W,), jnp.int32),
            pltpu.VMEM((2, HIDDEN), jnp.float32),
            pltpu.VMEM((_NBUF, _CHUNK, HIDDEN), jnp.float32),
            pltpu.SemaphoreType.DMA((_NBUF,)),
        ],
    )
    def embed(table_hbm, idx_hbm, out_hbm, idx_v, tab_v, rows_v, sem_s):
        wid = lax.axis_index("s") * _NC + lax.axis_index("c")
        base = wid * _B_PER_W
        pltpu.sync_copy(idx_hbm.at[pl.ds(base, _B_PER_W)], idx_v)
        pltpu.sync_copy(table_hbm, tab_v)

        scatters = [None] * _NCHUNK

        def build_chunk(c):
            buf = rows_v.at[c % _NBUF]
            for h in range(2):
                w0 = [tab_v[0, pl.ds((h * _HALF + dc) * _L, _L)]
                      for dc in range(_HALF)]
                diff = [tab_v[1, pl.ds((h * _HALF + dc) * _L, _L)] - w0[dc]
                        for dc in range(_HALF)]

                def body(b, carry):
                    grp = idx_v[pl.ds(c * _CHUNK + (b // _L) * _L, _L)]
                    f = grp.at[jnp.full((_L,), b % _L, jnp.int32)].get(
                        mode="promise_in_bounds").astype(jnp.float32)
                    for dc in range(_HALF):
                        buf[b, pl.ds((h * _HALF + dc) * _L, _L)] = (
                            w0[dc] + f * diff[dc])
                    return carry

                lax.fori_loop(0, _CHUNK, body, 0)

        def start_scatter(c):
            scatters[c] = pltpu.async_copy(
                rows_v.at[c % _NBUF],
                out_hbm.at[pl.ds(base + c * _CHUNK, _CHUNK)],
                sem_s.at[c % _NBUF])

        build_chunk(0)
        for c in range(_NCHUNK):
            start_scatter(c)
            if c + 1 < _NCHUNK:
                if c + 1 >= _NBUF:
                    scatters[c + 1 - _NBUF].wait()
                build_chunk(c + 1)
        for c in range(max(0, _NCHUNK - _NBUF), _NCHUNK):
            scatters[c].wait()

    return embed


_embed = _make_embed()


def kernel(domain_ids, embed_weight):
    ids = domain_ids.astype(jnp.int32)
    return _embed(embed_weight, ids)


# speedup vs baseline: 9.6160x; 1.0218x over previous
import functools

import jax
import jax.numpy as jnp
from jax import lax
from jax.experimental import pallas as pl
from jax.experimental.pallas import tpu as pltpu
from jax.experimental.pallas import tpu_sc as plsc

HIDDEN = 512
BATCH = 16384
_NC = 2
_NS = 16
_NW = _NC * _NS
_B_PER_W = BATCH // _NW
_CHUNK = 64
_NCHUNK = _B_PER_W // _CHUNK
_NBUF = 3
_L = 16
_DC = HIDDEN // _L
_HALF = _DC // 2


def _make_embed():
    mesh = plsc.VectorSubcoreMesh(core_axis_name="c", subcore_axis_name="s")

    @functools.partial(
        pl.kernel,
        mesh=mesh,
        out_type=jax.ShapeDtypeStruct((BATCH, HIDDEN), jnp.float32),
        scratch_types=[
            pltpu.VMEM((_B_PER_W,), jnp.int32),
            pltpu.VMEM((2, HIDDEN), jnp.float32),
            pltpu.VMEM((_NBUF, _CHUNK, HIDDEN), jnp.float32),
            pltpu.VMEM((_CHUNK, _L), jnp.float32),
            pltpu.SemaphoreType.DMA((_NBUF,)),
        ],
    )
    def embed(table_hbm, idx_hbm, out_hbm, idx_v, tab_v, rows_v, fvec_v,
              sem_s):
        wid = lax.axis_index("s") * _NC + lax.axis_index("c")
        base = wid * _B_PER_W
        pltpu.sync_copy(idx_hbm.at[pl.ds(base, _B_PER_W)], idx_v)
        pltpu.sync_copy(table_hbm, tab_v)

        scatters = [None] * _NCHUNK

        def build_chunk(c):
            buf = rows_v.at[c % _NBUF]

            def splat_body(g, carry):
                grpf = idx_v[pl.ds(c * _CHUNK + g * _L, _L)
                             ].astype(jnp.float32)
                for r in range(_L):
                    fvec_v[g * _L + r, :] = jnp.full((_L,), grpf[r])
                return carry

            lax.fori_loop(0, _CHUNK // _L, splat_body, 0)

            for h in range(2):
                w0 = [tab_v[0, pl.ds((h * _HALF + dc) * _L, _L)]
                      for dc in range(_HALF)]
                diff = [tab_v[1, pl.ds((h * _HALF + dc) * _L, _L)] - w0[dc]
                        for dc in range(_HALF)]

                def body(b, carry):
                    f = fvec_v[b, :]
                    for dc in range(_HALF):
                        buf[b, pl.ds((h * _HALF + dc) * _L, _L)] = (
                            w0[dc] + f * diff[dc])
                    return carry

                lax.fori_loop(0, _CHUNK, body, 0)

        def start_scatter(c):
            scatters[c] = pltpu.async_copy(
                rows_v.at[c % _NBUF],
                out_hbm.at[pl.ds(base + c * _CHUNK, _CHUNK)],
                sem_s.at[c % _NBUF])

        build_chunk(0)
        for c in range(_NCHUNK):
            start_scatter(c)
            if c + 1 < _NCHUNK:
                if c + 1 >= _NBUF:
                    scatters[c + 1 - _NBUF].wait()
                build_chunk(c + 1)
        for c in range(max(0, _NCHUNK - _NBUF), _NCHUNK):
            scatters[c].wait()

    return embed


_embed = _make_embed()


def kernel(domain_ids, embed_weight):
    ids = domain_ids.astype(jnp.int32)
    return _embed(embed_weight, ids)
